# Initial kernel scaffold; baseline (speedup 1.0000x reference)
import jax
import jax.numpy as jnp
from jax.experimental import pallas as pl


def kernel(x, edge_index, W1, b1, W2, b2):
    raise NotImplementedError("write your pallas kernel here")


# speedup vs baseline: 16.2190x; 16.2190x over previous
import functools

import jax
import jax.numpy as jnp
from jax import lax
from jax.experimental import pallas as pl
from jax.experimental.pallas import tpu as pltpu
from jax.experimental.pallas import tpu_sc as plsc

N = 50000
E = 1600000
D = 1433
H = 16
C = 5

NC = 2
NS = 16
NW = NC * NS
EW = E // NW
CH = 80
NCHUNK = EW // CH
NP = 51200
RW = NP // NS

@functools.lru_cache(maxsize=None)
def _sc_kernels():
    mesh = plsc.VectorSubcoreMesh(
        core_axis_name="c", subcore_axis_name="s", num_cores=NC, num_subcores=NS
    )

    @functools.partial(
        pl.kernel,
        out_type=jax.ShapeDtypeStruct((NC, NP), jnp.float32),
        mesh=mesh,
        scratch_types=[
            pltpu.VMEM((CH,), jnp.int32),
            pltpu.VMEM((CH,), jnp.float32),
            pltpu.VMEM_SHARED((NP,), jnp.float32),
        ],
    )
    def deg_kernel(dst_hbm, zero_hbm, out_hbm, idx_v, ones_v, acc_sh):
        c = lax.axis_index("c")
        s = lax.axis_index("s")
        wid = s * NC + c

        pltpu.sync_copy(zero_hbm.at[pl.ds(s * RW, RW)], acc_sh.at[pl.ds(s * RW, RW)])
        for i in range(CH // 16):
            ones_v[pl.ds(i * 16, 16)] = jnp.ones((16,), jnp.float32)
        plsc.subcore_barrier()

        def body(j, carry):
            base = wid * EW + j * CH
            pltpu.sync_copy(dst_hbm.at[pl.ds(base, CH)], idx_v)
            pltpu.sync_copy(ones_v, acc_sh.at[idx_v], add=True)
            return carry

        lax.fori_loop(0, NCHUNK, body, 0)
        plsc.subcore_barrier()

        pltpu.sync_copy(acc_sh.at[pl.ds(s * RW, RW)], out_hbm.at[c, pl.ds(s * RW, RW)])

    @functools.partial(
        pl.kernel,
        out_type=jax.ShapeDtypeStruct((NC, NP, H), jnp.float32),
        mesh=mesh,
        scratch_types=[
            pltpu.VMEM((CH,), jnp.int32),
            pltpu.VMEM((CH,), jnp.int32),
            pltpu.VMEM((CH, H), jnp.float32),
            pltpu.VMEM_SHARED((NP, H), jnp.float32),
            pltpu.SemaphoreType.DMA,
        ],
        compiler_params=pltpu.CompilerParams(use_tc_tiling_on_sc=False),
    )
    def msg_kernel(g_hbm, src_hbm, dst_hbm, zero_hbm, out_hbm,
                   sidx, didx, msg_v, acc_sh, sem):
        c = lax.axis_index("c")
        s = lax.axis_index("s")
        wid = s * NC + c

        pltpu.sync_copy(zero_hbm.at[pl.ds(s * RW, RW)], acc_sh.at[pl.ds(s * RW, RW)])
        plsc.subcore_barrier()

        def body(j, carry):
            base = wid * EW + j * CH
            pltpu.sync_copy(src_hbm.at[pl.ds(base, CH)], sidx)
            pltpu.sync_copy(dst_hbm.at[pl.ds(base, CH)], didx)
            pltpu.async_copy(g_hbm.at[sidx], msg_v, sem).wait()
            pltpu.sync_copy(msg_v, acc_sh.at[didx], add=True)
            return carry

        lax.fori_loop(0, NCHUNK, body, 0)
        plsc.subcore_barrier()

        pltpu.sync_copy(acc_sh.at[pl.ds(s * RW, RW)], out_hbm.at[c, pl.ds(s * RW, RW)])

    return deg_kernel, msg_kernel


def _mm_body(x_ref, w_ref, d0_ref, d1_ref, g_ref, dinv_ref):
    deg = d0_ref[...] + d1_ref[...] + 1.0
    dinv = lax.rsqrt(deg)
    h = jnp.dot(x_ref[...], w_ref[...], preferred_element_type=jnp.float32)
    g_ref[...] = h * dinv
    dinv_ref[...] = dinv


def _mid_body(a0_ref, a1_ref, g1_ref, dinv_ref, b1_ref, g2_ref):
    dinv = dinv_ref[...]
    out1 = (a0_ref[...] + a1_ref[...] + g1_ref[...]) * dinv + b1_ref[...]
    g2_ref[...] = jnp.maximum(out1, 0.0) * dinv


def _fin_body(a0_ref, a1_ref, g2_ref, dinv_ref, w2_ref, b2_ref, out_ref):
    agg = (a0_ref[...] + a1_ref[...] + g2_ref[...]) * dinv_ref[...]
    o = jnp.dot(agg, w2_ref[...], preferred_element_type=jnp.float32) + b2_ref[...]
    lane = lax.broadcasted_iota(jnp.int32, o.shape, 1)
    valid = lane < C
    o = jnp.where(valid, o, -1e30)
    m = jnp.max(o, axis=1, keepdims=True)
    e = jnp.where(valid, jnp.exp(o - m), 0.0)
    lse = jnp.log(jnp.sum(e, axis=1, keepdims=True))
    out_ref[...] = (o - m - lse)[:, :C]


def kernel(x, edge_index, W1, b1, W2, b2):
    edge_index = edge_index.astype(jnp.int32)
    src = edge_index[0]
    dst = edge_index[1]

    zeros_1d = jnp.zeros((NP,), jnp.float32)
    zeros_2d = jnp.zeros((NP, H), jnp.float32)

    deg_k, msg_k = _sc_kernels()
    degp = deg_k(dst, zeros_1d)
    d0 = degp[0, :N, None]
    d1 = degp[1, :N, None]

    BM = 1000
    g1, dinv = pl.pallas_call(
        _mm_body,
        grid=(N // BM,),
        in_specs=[
            pl.BlockSpec((BM, D), lambda i: (i, 0)),
            pl.BlockSpec((D, H), lambda i: (0, 0)),
            pl.BlockSpec((BM, 1), lambda i: (i, 0)),
            pl.BlockSpec((BM, 1), lambda i: (i, 0)),
        ],
        out_specs=[
            pl.BlockSpec((BM, H), lambda i: (i, 0)),
            pl.BlockSpec((BM, 1), lambda i: (i, 0)),
        ],
        out_shape=[
            jax.ShapeDtypeStruct((N, H), jnp.float32),
            jax.ShapeDtypeStruct((N, 1), jnp.float32),
        ],
    )(x, W1, d0, d1)

    acc1 = msg_k(g1, src, dst, zeros_2d)

    BE = 2000
    g2 = pl.pallas_call(
        _mid_body,
        grid=(N // BE,),
        in_specs=[
            pl.BlockSpec((BE, H), lambda i: (i, 0)),
            pl.BlockSpec((BE, H), lambda i: (i, 0)),
            pl.BlockSpec((BE, H), lambda i: (i, 0)),
            pl.BlockSpec((BE, 1), lambda i: (i, 0)),
            pl.BlockSpec((1, H), lambda i: (0, 0)),
        ],
        out_specs=pl.BlockSpec((BE, H), lambda i: (i, 0)),
        out_shape=jax.ShapeDtypeStruct((N, H), jnp.float32),
    )(acc1[0, :N], acc1[1, :N], g1, dinv, b1[None, :])

    acc2 = msg_k(g2, src, dst, zeros_2d)

    W2p = jnp.zeros((H, 128), jnp.float32).at[:, :C].set(W2)
    b2p = jnp.zeros((1, 128), jnp.float32).at[:, :C].set(b2)
    out = pl.pallas_call(
        _fin_body,
        grid=(N // BE,),
        in_specs=[
            pl.BlockSpec((BE, H), lambda i: (i, 0)),
            pl.BlockSpec((BE, H), lambda i: (i, 0)),
            pl.BlockSpec((BE, H), lambda i: (i, 0)),
            pl.BlockSpec((BE, 1), lambda i: (i, 0)),
            pl.BlockSpec((H, 128), lambda i: (0, 0)),
            pl.BlockSpec((1, 128), lambda i: (0, 0)),
        ],
        out_specs=pl.BlockSpec((BE, C), lambda i: (i, 0)),
        out_shape=jax.ShapeDtypeStruct((N, C), jnp.float32),
    )(acc2[0, :N], acc2[1, :N], g2, dinv, W2p, b2p)

    return out


# speedup vs baseline: 45.2484x; 2.7898x over previous
import functools

import jax
import jax.numpy as jnp
from jax import lax
from jax.experimental import pallas as pl
from jax.experimental.pallas import tpu as pltpu
from jax.experimental.pallas import tpu_sc as plsc

N = 50000
E = 1600000
D = 1433
H = 16
C = 5

NC = 2
NS = 16
NW = NC * NS
EW = E // NW
CH = 2000
NCHUNK = EW // CH
NP = 51200
RW = NP // NS

@functools.lru_cache(maxsize=None)
def _sc_kernels():
    mesh = plsc.VectorSubcoreMesh(
        core_axis_name="c", subcore_axis_name="s", num_cores=NC, num_subcores=NS
    )

    @functools.partial(
        pl.kernel,
        out_type=jax.ShapeDtypeStruct((NC, NP), jnp.float32),
        mesh=mesh,
        scratch_types=[
            pltpu.VMEM((CH,), jnp.int32),
            pltpu.VMEM((CH,), jnp.float32),
            pltpu.VMEM_SHARED((NP,), jnp.float32),
        ],
    )
    def deg_kernel(dst_hbm, zero_hbm, out_hbm, idx_v, ones_v, acc_sh):
        c = lax.axis_index("c")
        s = lax.axis_index("s")
        wid = s * NC + c

        pltpu.sync_copy(zero_hbm.at[pl.ds(s * RW, RW)], acc_sh.at[pl.ds(s * RW, RW)])
        for i in range(CH // 16):
            ones_v[pl.ds(i * 16, 16)] = jnp.ones((16,), jnp.float32)
        plsc.subcore_barrier()

        def body(j, carry):
            base = wid * EW + j * CH
            pltpu.sync_copy(dst_hbm.at[pl.ds(base, CH)], idx_v)
            pltpu.sync_copy(ones_v, acc_sh.at[idx_v], add=True)
            return carry

        lax.fori_loop(0, NCHUNK, body, 0)
        plsc.subcore_barrier()

        pltpu.sync_copy(acc_sh.at[pl.ds(s * RW, RW)], out_hbm.at[c, pl.ds(s * RW, RW)])

    @functools.partial(
        pl.kernel,
        out_type=jax.ShapeDtypeStruct((NC, NP, H), jnp.float32),
        mesh=mesh,
        scratch_types=[
            pltpu.VMEM((CH,), jnp.int32),
            pltpu.VMEM((CH,), jnp.int32),
            pltpu.VMEM((CH, H), jnp.float32),
            pltpu.VMEM_SHARED((NP, H), jnp.float32),
            pltpu.SemaphoreType.DMA,
        ],
        compiler_params=pltpu.CompilerParams(use_tc_tiling_on_sc=False),
    )
    def msg_kernel(g_hbm, src_hbm, dst_hbm, zero_hbm, out_hbm,
                   sidx, didx, msg_v, acc_sh, sem):
        c = lax.axis_index("c")
        s = lax.axis_index("s")
        wid = s * NC + c

        pltpu.sync_copy(zero_hbm.at[pl.ds(s * RW, RW)], acc_sh.at[pl.ds(s * RW, RW)])
        plsc.subcore_barrier()

        def body(j, carry):
            base = wid * EW + j * CH
            pltpu.sync_copy(src_hbm.at[pl.ds(base, CH)], sidx)
            pltpu.sync_copy(dst_hbm.at[pl.ds(base, CH)], didx)
            pltpu.async_copy(g_hbm.at[sidx], msg_v, sem).wait()
            pltpu.sync_copy(msg_v, acc_sh.at[didx], add=True)
            return carry

        lax.fori_loop(0, NCHUNK, body, 0)
        plsc.subcore_barrier()

        pltpu.sync_copy(acc_sh.at[pl.ds(s * RW, RW)], out_hbm.at[c, pl.ds(s * RW, RW)])

    return deg_kernel, msg_kernel


def _mm_body(x_ref, w_ref, d0_ref, d1_ref, g_ref, dinv_ref):
    deg = d0_ref[...] + d1_ref[...] + 1.0
    dinv = lax.rsqrt(deg)
    h = jnp.dot(x_ref[...], w_ref[...], preferred_element_type=jnp.float32)
    g_ref[...] = h * dinv
    dinv_ref[...] = dinv


def _mid_body(a0_ref, a1_ref, g1_ref, dinv_ref, b1_ref, g2_ref):
    dinv = dinv_ref[...]
    out1 = (a0_ref[...] + a1_ref[...] + g1_ref[...]) * dinv + b1_ref[...]
    g2_ref[...] = jnp.maximum(out1, 0.0) * dinv


def _fin_body(a0_ref, a1_ref, g2_ref, dinv_ref, w2_ref, b2_ref, out_ref):
    agg = (a0_ref[...] + a1_ref[...] + g2_ref[...]) * dinv_ref[...]
    o = jnp.dot(agg, w2_ref[...], preferred_element_type=jnp.float32) + b2_ref[...]
    lane = lax.broadcasted_iota(jnp.int32, o.shape, 1)
    valid = lane < C
    o = jnp.where(valid, o, -1e30)
    m = jnp.max(o, axis=1, keepdims=True)
    e = jnp.where(valid, jnp.exp(o - m), 0.0)
    lse = jnp.log(jnp.sum(e, axis=1, keepdims=True))
    out_ref[...] = (o - m - lse)[:, :C]


def kernel(x, edge_index, W1, b1, W2, b2):
    edge_index = edge_index.astype(jnp.int32)
    src = edge_index[0]
    dst = edge_index[1]

    zeros_1d = jnp.zeros((NP,), jnp.float32)
    zeros_2d = jnp.zeros((NP, H), jnp.float32)

    deg_k, msg_k = _sc_kernels()
    degp = deg_k(dst, zeros_1d)
    d0 = degp[0, :N, None]
    d1 = degp[1, :N, None]

    BM = 1000
    g1, dinv = pl.pallas_call(
        _mm_body,
        grid=(N // BM,),
        in_specs=[
            pl.BlockSpec((BM, D), lambda i: (i, 0)),
            pl.BlockSpec((D, H), lambda i: (0, 0)),
            pl.BlockSpec((BM, 1), lambda i: (i, 0)),
            pl.BlockSpec((BM, 1), lambda i: (i, 0)),
        ],
        out_specs=[
            pl.BlockSpec((BM, H), lambda i: (i, 0)),
            pl.BlockSpec((BM, 1), lambda i: (i, 0)),
        ],
        out_shape=[
            jax.ShapeDtypeStruct((N, H), jnp.float32),
            jax.ShapeDtypeStruct((N, 1), jnp.float32),
        ],
    )(x, W1, d0, d1)

    acc1 = msg_k(g1, src, dst, zeros_2d)

    BE = 2000
    g2 = pl.pallas_call(
        _mid_body,
        grid=(N // BE,),
        in_specs=[
            pl.BlockSpec((BE, H), lambda i: (i, 0)),
            pl.BlockSpec((BE, H), lambda i: (i, 0)),
            pl.BlockSpec((BE, H), lambda i: (i, 0)),
            pl.BlockSpec((BE, 1), lambda i: (i, 0)),
            pl.BlockSpec((1, H), lambda i: (0, 0)),
        ],
        out_specs=pl.BlockSpec((BE, H), lambda i: (i, 0)),
        out_shape=jax.ShapeDtypeStruct((N, H), jnp.float32),
    )(acc1[0, :N], acc1[1, :N], g1, dinv, b1[None, :])

    acc2 = msg_k(g2, src, dst, zeros_2d)

    W2p = jnp.zeros((H, 128), jnp.float32).at[:, :C].set(W2)
    b2p = jnp.zeros((1, 128), jnp.float32).at[:, :C].set(b2)
    out = pl.pallas_call(
        _fin_body,
        grid=(N // BE,),
        in_specs=[
            pl.BlockSpec((BE, H), lambda i: (i, 0)),
            pl.BlockSpec((BE, H), lambda i: (i, 0)),
            pl.BlockSpec((BE, H), lambda i: (i, 0)),
            pl.BlockSpec((BE, 1), lambda i: (i, 0)),
            pl.BlockSpec((H, 128), lambda i: (0, 0)),
            pl.BlockSpec((1, 128), lambda i: (0, 0)),
        ],
        out_specs=pl.BlockSpec((BE, C), lambda i: (i, 0)),
        out_shape=jax.ShapeDtypeStruct((N, C), jnp.float32),
    )(acc2[0, :N], acc2[1, :N], g2, dinv, W2p, b2p)

    return out
